# baseline (speedup 1.0000x reference)
import jax
import jax.numpy as jnp
from jax.experimental import pallas as pl
from jax.experimental.pallas import tpu as pltpu

_BPB = 4


def _body(l_ref, u_ref, o_ref, r_ref):
    K = u_ref.shape[1]
    for j in range(_BPB):
        e = jnp.exp(2.0 * l_ref[j])
        t = jnp.log(u_ref[j])
        r = (1.0 / (t * t)) * e
        rj = r_ref.at[pl.ds(j * K, K), :]
        rj[...] = r
        s = jnp.sum(r, axis=1, keepdims=True)
        o_ref[j] = jnp.max(rj[...] * (1.0 / s), axis=0, keepdims=True)


def kernel(logits, uniform):
    B, K, D = uniform.shape
    out = pl.pallas_call(
        _body,
        grid=(B // _BPB,),
        in_specs=[
            pl.BlockSpec((_BPB, 1, D), lambda b: (b, 0, 0)),
            pl.BlockSpec((_BPB, K, D), lambda b: (b, 0, 0)),
        ],
        out_specs=pl.BlockSpec((_BPB, 1, D), lambda b: (b, 0, 0)),
        out_shape=jax.ShapeDtypeStruct((B, 1, D), jnp.float32),
        scratch_shapes=[pltpu.VMEM((_BPB * K, D), jnp.float32)],
    )(logits.reshape(B, 1, D), uniform)
    return out.reshape(B, D)
